# speedup vs baseline: 1.0109x; 1.0109x over previous
import functools
import jax
import jax.numpy as jnp
from jax import lax
from jax.experimental import pallas as pl
from jax.experimental.pallas import tpu as pltpu
from jax.experimental.pallas import tpu_sc as plsc

NC = 2
NS = 16
L = 16
NW = NC * NS

B, SEQ, D = 4, 8192, 1024
N = B * SEQ
ROWS_PER_W = N // NW
K = 16
NCHUNK = ROWS_PER_W // K


def _body(x_hbm, ids_hbm, emb_hbm, out_hbm, idx_v, x_v, emb_v, gsem):
  wid = lax.axis_index("s") * NC + lax.axis_index("c")
  base = wid * ROWS_PER_W
  pltpu.sync_copy(ids_hbm.at[pl.ds(base, ROWS_PER_W)], idx_v)

  def chunk(g, carry):
    rbase = base + g * K
    gather = pltpu.async_copy(
        emb_hbm.at[idx_v.at[pl.ds(g * K, K)]], emb_v, gsem)
    pltpu.sync_copy(x_hbm.at[pl.ds(rbase, K)], x_v)
    gather.wait()

    def row(r, c2):
      for c in range(D // L):
        sl = pl.ds(c * L, L)
        x_v[r, sl] = x_v[r, sl] + emb_v[r, sl]
      return c2

    lax.fori_loop(0, K, row, 0)
    pltpu.sync_copy(x_v, out_hbm.at[pl.ds(rbase, K)])
    return carry

  lax.fori_loop(0, NCHUNK, chunk, 0)


@jax.jit
def _run(x2d, ids, emb):
  mesh = plsc.VectorSubcoreMesh(
      core_axis_name="c", subcore_axis_name="s", num_cores=NC,
      num_subcores=NS)
  f = pl.kernel(
      _body,
      out_type=jax.ShapeDtypeStruct((N, D), jnp.float32),
      mesh=mesh,
      scratch_types=[
          pltpu.VMEM((ROWS_PER_W,), jnp.int32),
          pltpu.VMEM((K, D), jnp.float32),
          pltpu.VMEM((K, D), jnp.float32),
          pltpu.SemaphoreType.DMA,
      ],
  )
  return f(x2d, ids, emb)


def kernel(x, position_ids, pos_emb):
  x2d = x.reshape(N, D)
  ids = position_ids.astype(jnp.int32).reshape(N)
  out = _run(x2d, ids, pos_emb)
  return out.reshape(x.shape)


# speedup vs baseline: 1.9184x; 1.8977x over previous
import jax
import jax.numpy as jnp
from jax import lax
from jax.experimental import pallas as pl
from jax.experimental.pallas import tpu as pltpu
from jax.experimental.pallas import tpu_sc as plsc

NC = 2
NS = 16
L = 16
NW = NC * NS

B, SEQ, D = 4, 8192, 1024
N = B * SEQ
ROWS_PER_W = N // NW
K = 8
NCHUNK = ROWS_PER_W // K
NBUF = 4
PF = 2


def _body(x_hbm, ids_hbm, emb_hbm, out_hbm, idx_v, xb, eb, gs, xs, osem):
  wid = lax.axis_index("s") * NC + lax.axis_index("c")
  base = wid * ROWS_PER_W
  pltpu.sync_copy(ids_hbm.at[pl.ds(base, ROWS_PER_W)], idx_v)

  def start_in(g, b):
    pltpu.async_copy(emb_hbm.at[idx_v.at[pl.ds(g * K, K)]], eb[b], gs[b])
    pltpu.async_copy(x_hbm.at[pl.ds(base + g * K, K)], xb[b], xs[b])

  def wait_in(b):
    pltpu.make_async_copy(x_hbm.at[pl.ds(0, K)], eb[b], gs[b]).wait()
    pltpu.make_async_copy(x_hbm.at[pl.ds(0, K)], xb[b], xs[b]).wait()

  def wait_out(b):
    pltpu.make_async_copy(x_hbm.at[pl.ds(0, K)], eb[b], osem[b]).wait()

  start_in(0, 0)
  start_in(1, 1)

  def outer(go, carry):
    for j in range(NBUF):
      g = go * NBUF + j
      gp = g + PF
      bp = (j + PF) % NBUF

      @pl.when(gp < NCHUNK)
      def _prefetch():
        @pl.when(gp >= NBUF)
        def _drain():
          wait_out(bp)
        start_in(gp, bp)

      wait_in(j)

      def row(r, c2, j=j):
        for c in range(D // L):
          sl = pl.ds(c * L, L)
          plsc.addupdate(eb[j].at[r, sl], xb[j][r, sl])
        return c2

      lax.fori_loop(0, K, row, 0)
      pltpu.async_copy(eb[j], out_hbm.at[pl.ds(base + g * K, K)], osem[j])
    return carry

  lax.fori_loop(0, NCHUNK // NBUF, outer, 0)
  for j in range(NBUF):
    wait_out(j)


@jax.jit
def _run(x2d, ids, emb):
  mesh = plsc.VectorSubcoreMesh(
      core_axis_name="c", subcore_axis_name="s", num_cores=NC,
      num_subcores=NS)
  f = pl.kernel(
      _body,
      out_type=jax.ShapeDtypeStruct((N, D), jnp.float32),
      mesh=mesh,
      scratch_types=[
          pltpu.VMEM((ROWS_PER_W,), jnp.int32),
          [pltpu.VMEM((K, D), jnp.float32) for _ in range(NBUF)],
          [pltpu.VMEM((K, D), jnp.float32) for _ in range(NBUF)],
          [pltpu.SemaphoreType.DMA for _ in range(NBUF)],
          [pltpu.SemaphoreType.DMA for _ in range(NBUF)],
          [pltpu.SemaphoreType.DMA for _ in range(NBUF)],
      ],
  )
  return f(x2d, ids, emb)


def kernel(x, position_ids, pos_emb):
  x2d = x.reshape(N, D)
  ids = position_ids.astype(jnp.int32).reshape(N)
  out = _run(x2d, ids, pos_emb)
  return out.reshape(x.shape)
